# Initial kernel scaffold; baseline (speedup 1.0000x reference)
import jax
import jax.numpy as jnp
from jax.experimental import pallas as pl


def kernel(geom, price, liq, edge_index, params):
    raise NotImplementedError("write your pallas kernel here")


# speedup vs baseline: 3.3533x; 3.3533x over previous
import dataclasses
import functools

import jax
import jax.numpy as jnp
from jax import lax
from jax.experimental import pallas as pl
from jax.experimental.pallas import tpu as pltpu
from jax.experimental.pallas import tpu_sc as plsc

N = 10000
E = 320000
D = 128
H = 256
N_ENC = 3
N_GNN = 3

NC, NS = 2, 16
NW = NC * NS
K = 128
CPW = (E + NW * K - 1) // (NW * K)
CHUNKS = CPW * NW
E_PAD = CHUNKS * K
N_PAD = 10240
RPS = N_PAD // NS
ZR = 64

BLK = 1024
NB = N_PAD // BLK
HB = 1000
NHB = N // HB


def _ln(x):
    m = jnp.mean(x, axis=-1, keepdims=True)
    v = jnp.mean((x - m) ** 2, axis=-1, keepdims=True)
    return (x - m) / jnp.sqrt(v + 1e-5)


def _sc_aggregate(table, srcm, dstm, with_deg):
    mesh = plsc.VectorSubcoreMesh(core_axis_name="c", subcore_axis_name="s")
    w16 = D // 16

    out_type = [jax.ShapeDtypeStruct((NC, N_PAD, D), jnp.float32)]
    if with_deg:
        out_type.append(jax.ShapeDtypeStruct((NW, N_PAD), jnp.float32))
    scratch = [
        pltpu.VMEM_SHARED((N_PAD, D), jnp.float32),
        pltpu.VMEM((2, K), jnp.int32),
        pltpu.VMEM((2, K), jnp.int32),
        pltpu.VMEM((1, K, D), jnp.float32),
        pltpu.VMEM((ZR, D), jnp.float32),
        pltpu.SemaphoreType.DMA,
    ]
    if with_deg:
        scratch.append(pltpu.VMEM((N_PAD,), jnp.float32))

    cp = pltpu.CompilerParams()
    if "needs_layout_passes" in pltpu.CompilerParams.__dataclass_fields__:
        cp = dataclasses.replace(cp, needs_layout_passes=False)

    @functools.partial(pl.kernel, out_type=tuple(out_type), mesh=mesh,
                       scratch_types=scratch, compiler_params=cp)
    def agg_kernel(table_hbm, src_hbm, dst_hbm, out_hbm, *rest):
        if with_deg:
            deg_hbm, acc, sidx, didx, rows, zbuf, gsem, hist = rest
        else:
            acc, sidx, didx, rows, zbuf, gsem = rest
        c = lax.axis_index("c")
        s = lax.axis_index("s")
        wid = s * NC + c

        def zstore(i, _):
            r = i // w16
            col = (i % w16) * 16
            zbuf[r, pl.ds(col, 16)] = jnp.zeros((16,), jnp.float32)
            return 0
        lax.fori_loop(0, ZR * w16, zstore, 0)

        def zcopy(i, _):
            pltpu.sync_copy(zbuf, acc.at[pl.ds(s * RPS + i * ZR, ZR)])
            return 0
        lax.fori_loop(0, RPS // ZR, zcopy, 0)

        if with_deg:
            def zhist(i, _):
                hist[pl.ds(i * 16, 16)] = jnp.zeros((16,), jnp.float32)
                return 0
            lax.fori_loop(0, N_PAD // 16, zhist, 0)

        plsc.subcore_barrier()

        ones16 = jnp.ones((16,), jnp.float32)

        def chunk_body(g, _):
            chunk = wid * CPW + g
            pltpu.sync_copy(src_hbm.at[chunk], sidx.at[0])
            pltpu.sync_copy(dst_hbm.at[chunk], didx.at[0])
            pltpu.async_copy(table_hbm.at[sidx.at[0]], rows.at[0], gsem).wait()
            pltpu.sync_copy(rows.at[0], acc.at[didx.at[0]], add=True)
            if with_deg:
                for j in range(K // 16):
                    v = didx[0, pl.ds(j * 16, 16)]
                    plsc.addupdate_scatter(hist, [v], ones16)
            return 0
        lax.fori_loop(0, CPW, chunk_body, 0)

        plsc.subcore_barrier()

        pltpu.sync_copy(acc.at[pl.ds(s * RPS, RPS)],
                        out_hbm.at[c].at[pl.ds(s * RPS, RPS)])
        if with_deg:
            pltpu.sync_copy(hist, deg_hbm.at[wid])

    return agg_kernel(table, srcm, dstm)


def _full(shape):
    return pl.BlockSpec(shape, lambda i: (0,) * len(shape))


def _encoder_call(x16, w_in, b_in, enc_w):
    def body(x_ref, win_ref, bin_ref,
             w10, b10, w20, b20, w11, b11, w21, b21, w12, b12, w22, b22,
             out_ref):
        h = jnp.dot(x_ref[...], win_ref[...],
                    preferred_element_type=jnp.float32) + bin_ref[...]
        for (w1, b1, w2, b2) in ((w10, b10, w20, b20),
                                 (w11, b11, w21, b21),
                                 (w12, b12, w22, b22)):
            z = _ln(h)
            u = jax.nn.gelu(jnp.dot(z, w1[...],
                                    preferred_element_type=jnp.float32) + b1[...])
            h = h + jnp.dot(u, w2[...],
                            preferred_element_type=jnp.float32) + b2[...]
        out_ref[...] = h

    in_specs = [pl.BlockSpec((BLK, 16), lambda i: (i, 0)),
                _full((16, D)), _full((1, D))]
    args = [x16, w_in, b_in]
    for (w1, b1, w2, b2) in enc_w:
        in_specs += [_full((D, H)), _full((1, H)), _full((H, D)), _full((1, D))]
        args += [w1, b1, w2, b2]
    return pl.pallas_call(
        body,
        grid=(NB,),
        in_specs=in_specs,
        out_specs=pl.BlockSpec((BLK, D), lambda i: (i, 0)),
        out_shape=jax.ShapeDtypeStruct((N_PAD, D), jnp.float32),
    )(*args)


def _gnn_layer0_call(table, parts, degp, w1a, w1b, b1, w2, b2):
    def body(t_ref, p_ref, d_ref, w1a_ref, w1b_ref, b1_ref, w2_ref, b2_ref,
             out_ref, rdeg_ref):
        deg = jnp.sum(d_ref[...], axis=0)
        rdeg = 1.0 / jnp.maximum(deg, 1.0)
        agg = (p_ref[0] + p_ref[1]) * rdeg
        h = t_ref[...]
        u = (jnp.dot(h, w1a_ref[...], preferred_element_type=jnp.float32)
             + jnp.dot(agg, w1b_ref[...], preferred_element_type=jnp.float32)
             + b1_ref[...])
        hn = h + jnp.dot(jax.nn.gelu(u), w2_ref[...],
                         preferred_element_type=jnp.float32) + b2_ref[...]
        out_ref[...] = _ln(hn)
        rdeg_ref[...] = jnp.broadcast_to(rdeg, (BLK, D))

    return pl.pallas_call(
        body,
        grid=(NB,),
        in_specs=[pl.BlockSpec((BLK, D), lambda i: (i, 0)),
                  pl.BlockSpec((NC, BLK, D), lambda i: (0, i, 0)),
                  pl.BlockSpec((NW, BLK, 1), lambda i: (0, i, 0)),
                  _full((D, H)), _full((D, H)), _full((1, H)),
                  _full((H, D)), _full((1, D))],
        out_specs=[pl.BlockSpec((BLK, D), lambda i: (i, 0)),
                   pl.BlockSpec((BLK, D), lambda i: (i, 0))],
        out_shape=[jax.ShapeDtypeStruct((N_PAD, D), jnp.float32),
                   jax.ShapeDtypeStruct((N_PAD, D), jnp.float32)],
    )(table, parts, degp, w1a, w1b, b1, w2, b2)


def _gnn_layer_call(table, parts, rdeg, w1a, w1b, b1, w2, b2):
    def body(t_ref, p_ref, rdeg_ref, w1a_ref, w1b_ref, b1_ref, w2_ref, b2_ref,
             out_ref):
        agg = (p_ref[0] + p_ref[1]) * rdeg_ref[...]
        h = t_ref[...]
        u = (jnp.dot(h, w1a_ref[...], preferred_element_type=jnp.float32)
             + jnp.dot(agg, w1b_ref[...], preferred_element_type=jnp.float32)
             + b1_ref[...])
        hn = h + jnp.dot(jax.nn.gelu(u), w2_ref[...],
                         preferred_element_type=jnp.float32) + b2_ref[...]
        out_ref[...] = _ln(hn)

    return pl.pallas_call(
        body,
        grid=(NB,),
        in_specs=[pl.BlockSpec((BLK, D), lambda i: (i, 0)),
                  pl.BlockSpec((NC, BLK, D), lambda i: (0, i, 0)),
                  pl.BlockSpec((BLK, D), lambda i: (i, 0)),
                  _full((D, H)), _full((D, H)), _full((1, H)),
                  _full((H, D)), _full((1, D))],
        out_specs=pl.BlockSpec((BLK, D), lambda i: (i, 0)),
        out_shape=jax.ShapeDtypeStruct((N_PAD, D), jnp.float32),
    )(table, parts, rdeg, w1a, w1b, b1, w2, b2)


def _heads_call(refined, wh, bh):
    def body(r_ref, wh_ref, bh_ref, heads_ref, mean_ref, max_ref, asum, amax):
        i = pl.program_id(0)
        r = r_ref[...]

        @pl.when(i == 0)
        def _init():
            asum[...] = jnp.zeros_like(asum)
            amax[...] = jnp.full_like(amax, -jnp.inf)

        asum[...] += jnp.sum(r, axis=0, keepdims=True)
        amax[...] = jnp.maximum(amax[...], jnp.max(r, axis=0, keepdims=True))
        heads_ref[...] = jnp.dot(r, wh_ref[...],
                                 preferred_element_type=jnp.float32) + bh_ref[...]

        @pl.when(i == NHB - 1)
        def _fin():
            mean_ref[...] = asum[...] / float(N)
            max_ref[...] = amax[...]

    return pl.pallas_call(
        body,
        grid=(NHB,),
        in_specs=[pl.BlockSpec((HB, D), lambda i: (i, 0)),
                  _full((D, D)), _full((1, D))],
        out_specs=[pl.BlockSpec((HB, D), lambda i: (i, 0)),
                   _full((1, D)), _full((1, D))],
        out_shape=[jax.ShapeDtypeStruct((N, D), jnp.float32),
                   jax.ShapeDtypeStruct((1, D), jnp.float32),
                   jax.ShapeDtypeStruct((1, D), jnp.float32)],
        scratch_shapes=[pltpu.VMEM((1, D), jnp.float32),
                        pltpu.VMEM((1, D), jnp.float32)],
    )(refined, wh, bh)


def kernel(geom, price, liq, edge_index, params):
    p = params
    f32 = jnp.float32

    x16 = jnp.zeros((N_PAD, 16), f32)
    x16 = x16.at[:N, 0:3].set(geom).at[:N, 3:7].set(price).at[:N, 7:11].set(liq)
    w_in = jnp.zeros((16, D), f32)
    w_in = w_in.at[0:3].set(p['Wg']).at[3:7].set(p['Wp']).at[7:11].set(p['Wl'])
    b_in = (p['bg'] + p['bp'] + p['bl']).reshape(1, D)

    enc_w = [(p[f'enc_W1_{i}'], p[f'enc_b1_{i}'].reshape(1, H),
              p[f'enc_W2_{i}'], p[f'enc_b2_{i}'].reshape(1, D))
             for i in range(N_ENC)]

    table0 = _encoder_call(x16, w_in, b_in, enc_w)

    src = edge_index[0]
    dst = edge_index[1]
    srcm = jnp.concatenate(
        [src, jnp.zeros((E_PAD - E,), jnp.int32)]).reshape(CHUNKS, K)
    dstm = jnp.concatenate(
        [dst, jnp.full((E_PAD - E,), N, jnp.int32)]).reshape(CHUNKS, K)

    gnn_w = []
    for i in range(N_GNN):
        w1 = p[f'gnn_W1_{i}']
        gnn_w.append((w1[:D], w1[D:], p[f'gnn_b1_{i}'].reshape(1, H),
                      p[f'gnn_W2_{i}'], p[f'gnn_b2_{i}'].reshape(1, D)))

    parts0, degp = _sc_aggregate(table0, srcm, dstm, with_deg=True)
    w1a, w1b, b1, w2, b2 = gnn_w[0]
    table, rdeg = _gnn_layer0_call(table0, parts0, degp.reshape(NW, N_PAD, 1),
                                   w1a, w1b, b1, w2, b2)

    for i in (1, 2):
        parts, = _sc_aggregate(table, srcm, dstm, with_deg=False)
        w1a, w1b, b1, w2, b2 = gnn_w[i]
        table = _gnn_layer_call(table, parts, rdeg, w1a, w1b, b1, w2, b2)

    refined = table[:N]

    wh = jnp.zeros((D, D), f32)
    wh = (wh.at[:, 0:1].set(p['iv_W']).at[:, 1:4].set(p['geom_W'])
            .at[:, 4:8].set(p['liq_W']).at[:, 8:13].set(p['greeks_W']))
    bh = jnp.zeros((1, D), f32)
    bh = (bh.at[0, 0:1].set(p['iv_b']).at[0, 1:4].set(p['geom_b'])
            .at[0, 4:8].set(p['liq_b']).at[0, 8:13].set(p['greeks_b']))

    heads, smean, smax = _heads_call(refined, wh, bh)
    surface_embedding = jnp.concatenate([smean[0], smax[0]], axis=-1)
    iv_pred = heads[:, 0:1]
    geom_recon = heads[:, 1:4]
    liq_recon = heads[:, 4:8]
    greeks_pred = heads[:, 8:13]
    return (refined, surface_embedding, iv_pred, geom_recon, liq_recon,
            greeks_pred)
